# baseline (speedup 1.0000x reference)
import functools
import math

import jax
import jax.numpy as jnp
from jax import lax
from jax.experimental import pallas as pl
from jax.experimental.pallas import tpu as pltpu
from jax.experimental.pallas import tpu_sc as plsc

TOPK = 32
RPAD = 64
NQ = 8192
NS = 8192
SUB = 16
NSUB = NS // SUB
NSUP = NS // 256
NUM_WORKERS = 32
ROWS_PER_W = NQ // NUM_WORKERS


def _proj_body(x_ref, wt_ref, o_ref):
    o_ref[...] = jnp.dot(x_ref[...], wt_ref[...],
                         preferred_element_type=jnp.float32)


def _project(x, wt, br=1024):
    n = x.shape[0]
    d = x.shape[1]
    return pl.pallas_call(
        _proj_body,
        grid=(n // br,),
        in_specs=[pl.BlockSpec((br, d), lambda i: (i, 0)),
                  pl.BlockSpec((d, RPAD), lambda i: (0, 0))],
        out_specs=pl.BlockSpec((br, RPAD), lambda i: (i, 0)),
        out_shape=jax.ShapeDtypeStruct((n, RPAD), jnp.float32),
    )(x, wt)


def _score_body(scale, rq_ref, rkt_ref, mask_ref, s_ref, sub_ref, sup_ref):
    s = jnp.dot(rq_ref[...], rkt_ref[...],
                preferred_element_type=jnp.float32)
    s = s * scale + mask_ref[...]
    s_ref[...] = s
    br = s.shape[0]
    sub = jnp.max(s.reshape(br, NSUB, SUB), axis=2)
    sub_ref[...] = sub
    sup_ref[...] = jnp.max(sub.reshape(br, NSUP, 16), axis=2)


def _scores(rq, rkt, mask2d, scale, br=128):
    nq = rq.shape[0]
    grid = nq // br
    return pl.pallas_call(
        functools.partial(_score_body, scale),
        grid=(grid,),
        in_specs=[pl.BlockSpec((br, RPAD), lambda i: (i, 0)),
                  pl.BlockSpec((RPAD, NS), lambda i: (0, 0)),
                  pl.BlockSpec((1, NS), lambda i: (0, 0))],
        out_specs=[pl.BlockSpec((br, NS), lambda i: (i, 0)),
                   pl.BlockSpec((br, NSUB), lambda i: (i, 0)),
                   pl.BlockSpec((br, NSUP), lambda i: (i, 0))],
        out_shape=[jax.ShapeDtypeStruct((nq, NS), jnp.float32),
                   jax.ShapeDtypeStruct((nq, NSUB), jnp.float32),
                   jax.ShapeDtypeStruct((nq, NSUP), jnp.float32)],
    )(rq, rkt, mask2d)


def _topk_body(rpw, scores_hbm, sub_hbm, sup_hbm, idx_hbm, val_hbm,
               row_a, row_b, row_c, row_d, sub_a, sub_b, sub_c, sub_d,
               m_all, idx_acc, val_acc, sem_a, sem_b, sem_c, sem_d):
    cc = lax.axis_index("c")
    ss = lax.axis_index("s")
    wid = ss * 2 + cc
    base = wid * rpw
    iota = lax.broadcasted_iota(jnp.int32, (16,), 0)
    lane0 = iota == 0
    NEG = jnp.float32(-jnp.inf)
    BIG = jnp.int32(1 << 30)
    NEG_VEC = jnp.full((16,), NEG, jnp.float32)

    def _putv(ref, r, pos_v, val_v):
        plsc.store_scatter(ref, [jnp.full((16,), r, jnp.int32), pos_v],
                           val_v, mask=lane0)

    def _shuf(x, s):
        return x.at[iota ^ s].get(mode="promise_in_bounds")

    def _lanemax(x):
        for sh in (8, 4, 2, 1):
            x = jnp.maximum(x, _shuf(x, sh))
        return x

    pltpu.sync_copy(sup_hbm.at[pl.ds(base, rpw)], m_all)
    pltpu.async_copy(scores_hbm.at[base], row_a, sem_a)
    pltpu.async_copy(sub_hbm.at[base], sub_a, sem_a)
    pltpu.async_copy(scores_hbm.at[base + 1], row_b, sem_b)
    pltpu.async_copy(sub_hbm.at[base + 1], sub_b, sem_b)

    def step(i, m2, r, row_v, sub_v):
        mm = jnp.maximum(m2[0], m2[1])
        cmax_v = _lanemax(mm)
        f0 = plsc.all_reduce_ffs(m2[0] == cmax_v)
        f1 = plsc.all_reduce_ffs(m2[1] == cmax_v)
        s_v = jnp.minimum(jnp.where(f0 < 16, f0, BIG), f1 + 16)
        l1 = sub_v[pl.ds(s_v[0] * 16, 16)]
        ft = plsc.all_reduce_ffs(l1 == cmax_v)
        t_v = s_v * 16 + ft
        x = row_v[pl.ds(t_v[0] * 16, 16)]
        fl = plsc.all_reduce_ffs(x == cmax_v)
        p_v = t_v * 16 + fl
        xm = jnp.where(iota == fl, NEG, x)
        l1m = jnp.where(iota == ft, NEG, l1)
        sm_v = _lanemax(xm)
        newsup = jnp.maximum(sm_v, _lanemax(l1m))
        plsc.store_scatter(row_v, [p_v], NEG_VEC, mask=lane0)
        plsc.store_scatter(sub_v, [t_v], sm_v, mask=lane0)
        ivec = jnp.full((16,), i, jnp.int32)
        _putv(idx_acc, r, ivec, p_v)
        _putv(val_acc, r, ivec, cmax_v)
        sdiv = s_v >> 4
        smod = s_v & 15
        return tuple(
            jnp.where((iota == smod) & (sdiv == j), newsup, m2[j])
            for j in range(2))

    def process_pair(r, row_x, row_y, sub_x, sub_y):
        def it_body(i, m):
            ma = step(i, m[:2], r, row_x, sub_x)
            mb = step(i, m[2:], r + 1, row_y, sub_y)
            return ma + mb

        m0 = tuple(m_all[r, pl.ds(16 * j, 16)] for j in range(2))
        m1 = tuple(m_all[r + 1, pl.ds(16 * j, 16)] for j in range(2))
        lax.fori_loop(0, TOPK, it_body, m0 + m1)

    def body4(q, carry):
        r0 = 4 * q
        pltpu.async_copy(scores_hbm.at[base + r0 + 2], row_c, sem_c)
        pltpu.async_copy(sub_hbm.at[base + r0 + 2], sub_c, sem_c)
        pltpu.async_copy(scores_hbm.at[base + r0 + 3], row_d, sem_d)
        pltpu.async_copy(sub_hbm.at[base + r0 + 3], sub_d, sem_d)
        pltpu.make_async_copy(scores_hbm.at[base + r0], row_a, sem_a).wait()
        pltpu.make_async_copy(sub_hbm.at[base + r0], sub_a, sem_a).wait()
        pltpu.make_async_copy(scores_hbm.at[base + r0 + 1], row_b, sem_b).wait()
        pltpu.make_async_copy(sub_hbm.at[base + r0 + 1], sub_b, sem_b).wait()
        process_pair(r0, row_a, row_b, sub_a, sub_b)

        @pl.when(q < rpw // 4 - 1)
        def _():
            pltpu.async_copy(scores_hbm.at[base + r0 + 4], row_a, sem_a)
            pltpu.async_copy(sub_hbm.at[base + r0 + 4], sub_a, sem_a)
            pltpu.async_copy(scores_hbm.at[base + r0 + 5], row_b, sem_b)
            pltpu.async_copy(sub_hbm.at[base + r0 + 5], sub_b, sem_b)

        pltpu.make_async_copy(scores_hbm.at[base + r0 + 2], row_c, sem_c).wait()
        pltpu.make_async_copy(sub_hbm.at[base + r0 + 2], sub_c, sem_c).wait()
        pltpu.make_async_copy(scores_hbm.at[base + r0 + 3], row_d, sem_d).wait()
        pltpu.make_async_copy(sub_hbm.at[base + r0 + 3], sub_d, sem_d).wait()
        process_pair(r0 + 2, row_c, row_d, sub_c, sub_d)
        return carry

    lax.fori_loop(0, rpw // 4, body4, 0)
    pltpu.sync_copy(idx_acc, idx_hbm.at[pl.ds(base, rpw)])
    pltpu.sync_copy(val_acc, val_hbm.at[pl.ds(base, rpw)])


def _topk(scores, sub, sup):
    nq = scores.shape[0]
    rpw = nq // NUM_WORKERS
    mesh = plsc.VectorSubcoreMesh(core_axis_name="c", subcore_axis_name="s")
    fn = pl.kernel(
        functools.partial(_topk_body, rpw),
        out_type=[jax.ShapeDtypeStruct((nq, TOPK), jnp.int32),
                  jax.ShapeDtypeStruct((nq, TOPK), jnp.float32)],
        mesh=mesh,
        compiler_params=pltpu.CompilerParams(needs_layout_passes=False),
        scratch_types=[pltpu.VMEM((NS,), jnp.float32),
                       pltpu.VMEM((NS,), jnp.float32),
                       pltpu.VMEM((NS,), jnp.float32),
                       pltpu.VMEM((NS,), jnp.float32),
                       pltpu.VMEM((NSUB,), jnp.float32),
                       pltpu.VMEM((NSUB,), jnp.float32),
                       pltpu.VMEM((NSUB,), jnp.float32),
                       pltpu.VMEM((NSUB,), jnp.float32),
                       pltpu.VMEM((rpw, NSUP), jnp.float32),
                       pltpu.VMEM((rpw, TOPK), jnp.int32),
                       pltpu.VMEM((rpw, TOPK), jnp.float32),
                       pltpu.SemaphoreType.DMA,
                       pltpu.SemaphoreType.DMA,
                       pltpu.SemaphoreType.DMA,
                       pltpu.SemaphoreType.DMA],
    )
    return fn(scores, sub, sup)


def kernel(query, slot_keys, reliability_mask, W_router):
    b, s, d = query.shape
    r = W_router.shape[0]
    scale = 1.0 / math.sqrt(r)
    q2 = query.reshape(b * s, d)
    wt = jnp.zeros((d, RPAD), jnp.float32).at[:, :r].set(W_router.T)
    rq = _project(q2, wt)
    rk = _project(slot_keys, wt)
    rkt = rk.T
    mask2d = reliability_mask.reshape(1, NS)
    ngroups = 4
    gsz = (b * s) // ngroups
    outs = []
    for g in range(ngroups):
        sc_g, sub_g, sup_g = _scores(rq[g * gsz:(g + 1) * gsz], rkt, mask2d,
                                     scale)
        outs.append(_topk(sc_g, sub_g, sup_g))
    idx = jnp.concatenate([o[0] for o in outs])
    val = jnp.concatenate([o[1] for o in outs])
    return idx.reshape(b, s, TOPK), val.reshape(b, s, TOPK)


# speedup vs baseline: 4.0646x; 4.0646x over previous
import functools
import math

import jax
import jax.numpy as jnp
from jax import lax
from jax.experimental import pallas as pl
from jax.experimental.pallas import tpu as pltpu
from jax.experimental.pallas import tpu_sc as plsc

TOPK = 32
RPAD = 64
NQ = 8192
NS = 8192
CHUNK = 128
NCHUNK = NS // CHUNK
NUM_WORKERS = 32
NGROUPS = 4


def _proj_body(x_ref, wt_ref, o_ref):
    o_ref[...] = jnp.dot(x_ref[...], wt_ref[...],
                         preferred_element_type=jnp.float32)


def _project(x, wt, br=1024):
    n = x.shape[0]
    d = x.shape[1]
    return pl.pallas_call(
        _proj_body,
        grid=(n // br,),
        in_specs=[pl.BlockSpec((br, d), lambda i: (i, 0)),
                  pl.BlockSpec((d, RPAD), lambda i: (0, 0))],
        out_specs=pl.BlockSpec((br, RPAD), lambda i: (i, 0)),
        out_shape=jax.ShapeDtypeStruct((n, RPAD), jnp.float32),
    )(x, wt)


def _score_body(scale, q_ref, wt_ref, rkt_ref, mask_ref, s_ref, cm_ref):
    rq = jnp.dot(q_ref[...], wt_ref[...], preferred_element_type=jnp.float32)
    s = jnp.dot(rq, rkt_ref[...], preferred_element_type=jnp.float32)
    s = s * scale + mask_ref[...]
    s_ref[...] = s
    br = s.shape[0]
    cm_ref[...] = jnp.max(s.reshape(br, NCHUNK, CHUNK), axis=2)


def _scores(q2, wt, rkt, mask2d, scale, br=256):
    nq = q2.shape[0]
    d = q2.shape[1]
    grid = nq // br
    return pl.pallas_call(
        functools.partial(_score_body, scale),
        grid=(grid,),
        in_specs=[pl.BlockSpec((br, d), lambda i: (i, 0)),
                  pl.BlockSpec((d, RPAD), lambda i: (0, 0)),
                  pl.BlockSpec((RPAD, NS), lambda i: (0, 0)),
                  pl.BlockSpec((1, NS), lambda i: (0, 0))],
        out_specs=[pl.BlockSpec((br, NS), lambda i: (i, 0)),
                   pl.BlockSpec((br, NCHUNK), lambda i: (i, 0))],
        out_shape=[jax.ShapeDtypeStruct((nq, NS), jnp.float32),
                   jax.ShapeDtypeStruct((nq, NCHUNK), jnp.float32)],
    )(q2, wt, rkt, mask2d)


def _topk_body(rpw, scores_hbm, cmax_hbm, idx_hbm, val_hbm,
               quad0, quad1, m_all, idx_acc, val_acc, sem0, sem1):
    cc = lax.axis_index("c")
    ss = lax.axis_index("s")
    wid = ss * 2 + cc
    base = wid * rpw
    iota = lax.broadcasted_iota(jnp.int32, (16,), 0)
    lane0 = iota == 0
    NEG = jnp.float32(-jnp.inf)
    BIG = jnp.int32(1 << 30)
    NEG_VEC = jnp.full((16,), NEG, jnp.float32)

    def _putv(ref, r, pos_v, val_v):
        plsc.store_scatter(ref, [jnp.full((16,), r, jnp.int32), pos_v],
                           val_v, mask=lane0)

    def _shuf(x, s):
        return x.at[iota ^ s].get(mode="promise_in_bounds")

    def _lanemax(x):
        for sh in (8, 4, 2, 1):
            x = jnp.maximum(x, _shuf(x, sh))
        return x

    pltpu.sync_copy(cmax_hbm.at[pl.ds(base, rpw)], m_all)
    pltpu.async_copy(scores_hbm.at[pl.ds(base, 4)], quad0, sem0)

    def step(i, m, r, quad, k):
        mmv = jnp.maximum(jnp.maximum(m[0], m[1]),
                          jnp.maximum(m[2], m[3]))
        cmax_v = _lanemax(mmv)
        cand = None
        for j in range(4):
            fj = plsc.all_reduce_ffs(m[j] == cmax_v)
            cj = jnp.where(fj < 16, fj + (16 * j), BIG)
            cand = cj if cand is None else jnp.minimum(cand, cj)
        cid_v = cand
        start = cid_v[0] * CHUNK
        xs, pos = [], None
        for j in range(8):
            x = quad[k, pl.ds(start + 16 * j, 16)]
            xs.append(x)
            fj = plsc.all_reduce_ffs(x == cmax_v)
            pj = jnp.where(fj < 16, (start + 16 * j) + fj, BIG)
            pos = pj if pos is None else jnp.minimum(pos, pj)
        p_v = pos
        nm = None
        for j in range(8):
            d = p_v - (start + 16 * j)
            xm = jnp.where(iota == d, NEG, xs[j])
            nm = xm if nm is None else jnp.maximum(nm, xm)
        newmax = _lanemax(nm)
        plsc.store_scatter(quad, [jnp.full((16,), k, jnp.int32), p_v],
                           NEG_VEC, mask=lane0)
        ivec = jnp.full((16,), i, jnp.int32)
        _putv(idx_acc, r, ivec, p_v)
        _putv(val_acc, r, ivec, cmax_v)
        cdiv = cid_v >> 4
        cmod = cid_v & 15
        return tuple(
            jnp.where((iota == cmod) & (cdiv == j), newmax, m[j])
            for j in range(4))

    def process_pair(r, quad, k):
        def it_body(i, m):
            ma = step(i, m[:4], r, quad, k)
            mb = step(i, m[4:], r + 1, quad, k + 1)
            return ma + mb

        m0 = tuple(m_all[r, pl.ds(16 * j, 16)] for j in range(4))
        m1 = tuple(m_all[r + 1, pl.ds(16 * j, 16)] for j in range(4))
        lax.fori_loop(0, TOPK, it_body, m0 + m1)

    nquads = rpw // 4

    def body2(q2i, carry):
        r0 = 8 * q2i
        pltpu.async_copy(scores_hbm.at[pl.ds(base + r0 + 4, 4)], quad1, sem1)
        pltpu.make_async_copy(scores_hbm.at[pl.ds(base + r0, 4)],
                              quad0, sem0).wait()
        process_pair(r0, quad0, 0)
        process_pair(r0 + 2, quad0, 2)

        @pl.when(q2i < nquads // 2 - 1)
        def _():
            pltpu.async_copy(scores_hbm.at[pl.ds(base + r0 + 8, 4)],
                             quad0, sem0)

        pltpu.make_async_copy(scores_hbm.at[pl.ds(base + r0 + 4, 4)],
                              quad1, sem1).wait()
        process_pair(r0 + 4, quad1, 0)
        process_pair(r0 + 6, quad1, 2)
        return carry

    lax.fori_loop(0, nquads // 2, body2, 0)
    pltpu.sync_copy(idx_acc, idx_hbm.at[pl.ds(base, rpw)])
    pltpu.sync_copy(val_acc, val_hbm.at[pl.ds(base, rpw)])


def _topk(scores, cmax):
    nq = scores.shape[0]
    rpw = nq // NUM_WORKERS
    mesh = plsc.VectorSubcoreMesh(core_axis_name="c", subcore_axis_name="s")
    fn = pl.kernel(
        functools.partial(_topk_body, rpw),
        out_type=[jax.ShapeDtypeStruct((nq, TOPK), jnp.int32),
                  jax.ShapeDtypeStruct((nq, TOPK), jnp.float32)],
        mesh=mesh,
        compiler_params=pltpu.CompilerParams(needs_layout_passes=False),
        scratch_types=[pltpu.VMEM((4, NS), jnp.float32),
                       pltpu.VMEM((4, NS), jnp.float32),
                       pltpu.VMEM((rpw, NCHUNK), jnp.float32),
                       pltpu.VMEM((rpw, TOPK), jnp.int32),
                       pltpu.VMEM((rpw, TOPK), jnp.float32),
                       pltpu.SemaphoreType.DMA,
                       pltpu.SemaphoreType.DMA],
    )
    return fn(scores, cmax)


def kernel(query, slot_keys, reliability_mask, W_router):
    b, s, d = query.shape
    r = W_router.shape[0]
    scale = 1.0 / math.sqrt(r)
    q2 = query.reshape(b * s, d)
    wt = jnp.zeros((d, RPAD), jnp.float32).at[:, :r].set(W_router.T)
    rk = _project(slot_keys, wt)
    rkt = rk.T
    mask2d = reliability_mask.reshape(1, NS)
    gsz = (b * s) // NGROUPS
    outs = []
    for g in range(NGROUPS):
        sc_g, cm_g = _scores(q2[g * gsz:(g + 1) * gsz], wt, rkt, mask2d,
                             scale)
        outs.append(_topk(sc_g, cm_g))
    idx = jnp.concatenate([o[0] for o in outs])
    val = jnp.concatenate([o[1] for o in outs])
    return idx.reshape(b, s, TOPK), val.reshape(b, s, TOPK)


# speedup vs baseline: 4.4378x; 1.0918x over previous
import functools
import math

import jax
import jax.numpy as jnp
from jax import lax
from jax.experimental import pallas as pl
from jax.experimental.pallas import tpu as pltpu
from jax.experimental.pallas import tpu_sc as plsc

TOPK = 32
RPAD = 64
NQ = 8192
NS = 8192
CHUNK = 128
NCHUNK = NS // CHUNK
NUM_WORKERS = 32
NGROUPS = 4


def _proj_body(x_ref, wt_ref, o_ref):
    o_ref[...] = jnp.dot(x_ref[...], wt_ref[...],
                         preferred_element_type=jnp.float32)


def _project(x, wt, br=1024):
    n = x.shape[0]
    d = x.shape[1]
    return pl.pallas_call(
        _proj_body,
        grid=(n // br,),
        in_specs=[pl.BlockSpec((br, d), lambda i: (i, 0)),
                  pl.BlockSpec((d, RPAD), lambda i: (0, 0))],
        out_specs=pl.BlockSpec((br, RPAD), lambda i: (i, 0)),
        out_shape=jax.ShapeDtypeStruct((n, RPAD), jnp.float32),
    )(x, wt)


def _score_body(scale, q_ref, wt_ref, rkt_ref, mask_ref, s_ref, cm_ref):
    rq = jnp.dot(q_ref[...], wt_ref[...], preferred_element_type=jnp.float32)
    s = jnp.dot(rq, rkt_ref[...], preferred_element_type=jnp.float32)
    s = s * scale + mask_ref[...]
    s_ref[...] = s
    br = s.shape[0]
    cm_ref[...] = jnp.max(s.reshape(br, NCHUNK, CHUNK), axis=2)


def _scores(q2, wt, rkt, mask2d, scale, br=256):
    nq = q2.shape[0]
    d = q2.shape[1]
    grid = nq // br
    return pl.pallas_call(
        functools.partial(_score_body, scale),
        grid=(grid,),
        in_specs=[pl.BlockSpec((br, d), lambda i: (i, 0)),
                  pl.BlockSpec((d, RPAD), lambda i: (0, 0)),
                  pl.BlockSpec((RPAD, NS), lambda i: (0, 0)),
                  pl.BlockSpec((1, NS), lambda i: (0, 0))],
        out_specs=[pl.BlockSpec((br, NS), lambda i: (i, 0)),
                   pl.BlockSpec((br, NCHUNK), lambda i: (i, 0))],
        out_shape=[jax.ShapeDtypeStruct((nq, NS), jnp.float32),
                   jax.ShapeDtypeStruct((nq, NCHUNK), jnp.float32)],
    )(q2, wt, rkt, mask2d)


def _topk_body(rpw, scores_hbm, cmax_hbm, idx_hbm, val_hbm,
               row_a, row_b, row_c, row_d, m_all, idx_acc, val_acc,
               sem_a, sem_b, sem_c, sem_d):
    cc = lax.axis_index("c")
    ss = lax.axis_index("s")
    wid = ss * 2 + cc
    base = wid * rpw
    iota = lax.broadcasted_iota(jnp.int32, (16,), 0)
    lane0 = iota == 0
    NEG = jnp.float32(-jnp.inf)
    BIG = jnp.int32(1 << 30)
    NEG_VEC = jnp.full((16,), NEG, jnp.float32)

    def _putv(ref, r, pos_v, val_v):
        plsc.store_scatter(ref, [jnp.full((16,), r, jnp.int32), pos_v],
                           val_v, mask=lane0)

    def _shuf(x, s):
        return x.at[iota ^ s].get(mode="promise_in_bounds")

    def _lanemax(x):
        for sh in (8, 4, 2, 1):
            x = jnp.maximum(x, _shuf(x, sh))
        return x

    pltpu.sync_copy(cmax_hbm.at[pl.ds(base, rpw)], m_all)
    pltpu.async_copy(scores_hbm.at[base], row_a, sem_a)
    pltpu.async_copy(scores_hbm.at[base + 1], row_b, sem_b)

    def step(i, m, r, row_v):
        mmv = jnp.maximum(jnp.maximum(m[0], m[1]),
                          jnp.maximum(m[2], m[3]))
        cmax_v = _lanemax(mmv)
        cand = None
        for j in range(4):
            fj = plsc.all_reduce_ffs(m[j] == cmax_v)
            cj = jnp.where(fj < 16, fj + (16 * j), BIG)
            cand = cj if cand is None else jnp.minimum(cand, cj)
        cid_v = cand
        start = cid_v[0] * CHUNK
        pos, am, dup = None, None, None
        for j in range(8):
            x = row_v[pl.ds(start + 16 * j, 16)]
            e = x == cmax_v
            fj = plsc.all_reduce_ffs(e)
            pj = jnp.where(fj < 16, (start + 16 * j) + fj, BIG)
            pos = pj if pos is None else jnp.minimum(pos, pj)
            xm = jnp.where(e, NEG, x)
            am = xm if am is None else jnp.maximum(am, xm)
            cj = plsc.all_reduce_population_count(e)
            dup = cj if dup is None else dup + cj
        p_v = pos
        newmax = jnp.where(dup > 1, cmax_v, _lanemax(am))
        plsc.store_scatter(row_v, [p_v], NEG_VEC, mask=lane0)
        ivec = jnp.full((16,), i, jnp.int32)
        _putv(idx_acc, r, ivec, p_v)
        _putv(val_acc, r, ivec, cmax_v)
        cdiv = cid_v >> 4
        cmod = cid_v & 15
        return tuple(
            jnp.where((iota == cmod) & (cdiv == j), newmax, m[j])
            for j in range(4))

    def process_pair(r, row_x, row_y):
        def it_body(i, m):
            ma = step(i, m[:4], r, row_x)
            mb = step(i, m[4:], r + 1, row_y)
            return ma + mb

        m0 = tuple(m_all[r, pl.ds(16 * j, 16)] for j in range(4))
        m1 = tuple(m_all[r + 1, pl.ds(16 * j, 16)] for j in range(4))
        lax.fori_loop(0, TOPK, it_body, m0 + m1)

    def body4(q, carry):
        r0 = 4 * q
        pltpu.async_copy(scores_hbm.at[base + r0 + 2], row_c, sem_c)
        pltpu.async_copy(scores_hbm.at[base + r0 + 3], row_d, sem_d)
        pltpu.make_async_copy(scores_hbm.at[base + r0], row_a, sem_a).wait()
        pltpu.make_async_copy(scores_hbm.at[base + r0 + 1], row_b, sem_b).wait()
        process_pair(r0, row_a, row_b)

        @pl.when(q < rpw // 4 - 1)
        def _():
            pltpu.async_copy(scores_hbm.at[base + r0 + 4], row_a, sem_a)
            pltpu.async_copy(scores_hbm.at[base + r0 + 5], row_b, sem_b)

        pltpu.make_async_copy(scores_hbm.at[base + r0 + 2], row_c, sem_c).wait()
        pltpu.make_async_copy(scores_hbm.at[base + r0 + 3], row_d, sem_d).wait()
        process_pair(r0 + 2, row_c, row_d)
        return carry

    lax.fori_loop(0, rpw // 4, body4, 0)
    pltpu.sync_copy(idx_acc, idx_hbm.at[pl.ds(base, rpw)])
    pltpu.sync_copy(val_acc, val_hbm.at[pl.ds(base, rpw)])


def _topk(scores, cmax):
    nq = scores.shape[0]
    rpw = nq // NUM_WORKERS
    mesh = plsc.VectorSubcoreMesh(core_axis_name="c", subcore_axis_name="s")
    fn = pl.kernel(
        functools.partial(_topk_body, rpw),
        out_type=[jax.ShapeDtypeStruct((nq, TOPK), jnp.int32),
                  jax.ShapeDtypeStruct((nq, TOPK), jnp.float32)],
        mesh=mesh,
        compiler_params=pltpu.CompilerParams(needs_layout_passes=False),
        scratch_types=[pltpu.VMEM((NS,), jnp.float32),
                       pltpu.VMEM((NS,), jnp.float32),
                       pltpu.VMEM((NS,), jnp.float32),
                       pltpu.VMEM((NS,), jnp.float32),
                       pltpu.VMEM((rpw, NCHUNK), jnp.float32),
                       pltpu.VMEM((rpw, TOPK), jnp.int32),
                       pltpu.VMEM((rpw, TOPK), jnp.float32),
                       pltpu.SemaphoreType.DMA,
                       pltpu.SemaphoreType.DMA,
                       pltpu.SemaphoreType.DMA,
                       pltpu.SemaphoreType.DMA],
    )
    return fn(scores, cmax)


def kernel(query, slot_keys, reliability_mask, W_router):
    b, s, d = query.shape
    r = W_router.shape[0]
    scale = 1.0 / math.sqrt(r)
    q2 = query.reshape(b * s, d)
    wt = jnp.zeros((d, RPAD), jnp.float32).at[:, :r].set(W_router.T)
    rk = _project(slot_keys, wt)
    rkt = rk.T
    mask2d = reliability_mask.reshape(1, NS)
    gsz = (b * s) // NGROUPS
    outs = []
    for g in range(NGROUPS):
        sc_g, cm_g = _scores(q2[g * gsz:(g + 1) * gsz], wt, rkt, mask2d,
                             scale)
        outs.append(_topk(sc_g, cm_g))
    idx = jnp.concatenate([o[0] for o in outs])
    val = jnp.concatenate([o[1] for o in outs])
    return idx.reshape(b, s, TOPK), val.reshape(b, s, TOPK)


# speedup vs baseline: 4.7250x; 1.0647x over previous
import functools
import math

import jax
import jax.numpy as jnp
from jax import lax
from jax.experimental import pallas as pl
from jax.experimental.pallas import tpu as pltpu
from jax.experimental.pallas import tpu_sc as plsc

TOPK = 32
RPAD = 64
NQ = 8192
NS = 8192
CHUNK = 128
NCHUNK = NS // CHUNK
NUM_WORKERS = 32
NGROUPS = 4


def _proj_body(x_ref, wt_ref, o_ref):
    o_ref[...] = jnp.dot(x_ref[...], wt_ref[...],
                         preferred_element_type=jnp.float32)


def _project(x, wt, br=1024):
    n = x.shape[0]
    d = x.shape[1]
    return pl.pallas_call(
        _proj_body,
        grid=(n // br,),
        in_specs=[pl.BlockSpec((br, d), lambda i: (i, 0)),
                  pl.BlockSpec((d, RPAD), lambda i: (0, 0))],
        out_specs=pl.BlockSpec((br, RPAD), lambda i: (i, 0)),
        out_shape=jax.ShapeDtypeStruct((n, RPAD), jnp.float32),
    )(x, wt)


def _score_body(scale, q_ref, wt_ref, rkt_ref, mask_ref, s_ref, cm_ref):
    rq = jnp.dot(q_ref[...], wt_ref[...], preferred_element_type=jnp.float32)
    s = jnp.dot(rq, rkt_ref[...], preferred_element_type=jnp.float32)
    s = s * scale + mask_ref[...]
    s_ref[...] = s
    br = s.shape[0]
    cm_ref[...] = jnp.max(s.reshape(br, NCHUNK, CHUNK), axis=2)


def _scores(q2, wt, rkt, mask2d, scale, br=256):
    nq = q2.shape[0]
    d = q2.shape[1]
    grid = nq // br
    return pl.pallas_call(
        functools.partial(_score_body, scale),
        grid=(grid,),
        in_specs=[pl.BlockSpec((br, d), lambda i: (i, 0)),
                  pl.BlockSpec((d, RPAD), lambda i: (0, 0)),
                  pl.BlockSpec((RPAD, NS), lambda i: (0, 0)),
                  pl.BlockSpec((1, NS), lambda i: (0, 0))],
        out_specs=[pl.BlockSpec((br, NS), lambda i: (i, 0)),
                   pl.BlockSpec((br, NCHUNK), lambda i: (i, 0))],
        out_shape=[jax.ShapeDtypeStruct((nq, NS), jnp.float32),
                   jax.ShapeDtypeStruct((nq, NCHUNK), jnp.float32)],
    )(q2, wt, rkt, mask2d)


def _topk_body(rpw, scores_hbm, cmax_hbm, idx_hbm, val_hbm,
               row_a, row_b, row_c, row_d, m_all, idx_acc, val_acc,
               sem_a, sem_b, sem_c, sem_d):
    cc = lax.axis_index("c")
    ss = lax.axis_index("s")
    wid = ss * 2 + cc
    base = wid * rpw
    iota = lax.broadcasted_iota(jnp.int32, (16,), 0)
    lane0 = iota == 0
    NEG = jnp.float32(-jnp.inf)
    BIG = jnp.int32(1 << 30)
    NEG_VEC = jnp.full((16,), NEG, jnp.float32)

    def _putv(ref, r, pos_v, val_v):
        plsc.store_scatter(ref, [jnp.full((16,), r, jnp.int32), pos_v],
                           val_v, mask=lane0)

    def _shuf(x, s):
        return x.at[iota ^ s].get(mode="promise_in_bounds")

    def _lanemax(x):
        for sh in (8, 4, 2, 1):
            x = jnp.maximum(x, _shuf(x, sh))
        return x

    pltpu.sync_copy(cmax_hbm.at[pl.ds(base, rpw)], m_all)
    pltpu.async_copy(scores_hbm.at[base], row_a, sem_a)
    pltpu.async_copy(scores_hbm.at[base + 1], row_b, sem_b)

    def step(i, m, r, row_v):
        mmv = jnp.maximum(jnp.maximum(m[0], m[1]),
                          jnp.maximum(m[2], m[3]))
        cmax_v = _lanemax(mmv)
        cand = None
        for j in range(4):
            fj = plsc.all_reduce_ffs(m[j] == cmax_v)
            cj = jnp.where(fj < 16, fj + (16 * j), BIG)
            cand = cj if cand is None else jnp.minimum(cand, cj)
        cid_v = cand
        start = cid_v[0] * CHUNK
        xs, pos = [], None
        for j in range(8):
            x = row_v[pl.ds(start + 16 * j, 16)]
            xs.append(x)
            fj = plsc.all_reduce_ffs(x == cmax_v)
            pj = jnp.where(fj < 16, (start + 16 * j) + fj, BIG)
            pos = pj if pos is None else jnp.minimum(pos, pj)
        p_v = pos
        nm = None
        for j in range(8):
            d = p_v - (start + 16 * j)
            xm = jnp.where(iota == d, NEG, xs[j])
            nm = xm if nm is None else jnp.maximum(nm, xm)
        newmax = jnp.max(nm)
        plsc.store_scatter(row_v, [p_v], NEG_VEC, mask=lane0)
        ivec = jnp.full((16,), i, jnp.int32)
        _putv(idx_acc, r, ivec, p_v)
        _putv(val_acc, r, ivec, cmax_v)
        cdiv = cid_v >> 4
        cmod = cid_v & 15
        return tuple(
            jnp.where((iota == cmod) & (cdiv == j), newmax, m[j])
            for j in range(4))

    def process_pair(r, row_x, row_y):
        def it_body(i, m):
            ma = step(i, m[:4], r, row_x)
            mb = step(i, m[4:], r + 1, row_y)
            return ma + mb

        m0 = tuple(m_all[r, pl.ds(16 * j, 16)] for j in range(4))
        m1 = tuple(m_all[r + 1, pl.ds(16 * j, 16)] for j in range(4))
        lax.fori_loop(0, TOPK, it_body, m0 + m1)

    def body4(q, carry):
        r0 = 4 * q
        pltpu.async_copy(scores_hbm.at[base + r0 + 2], row_c, sem_c)
        pltpu.async_copy(scores_hbm.at[base + r0 + 3], row_d, sem_d)
        pltpu.make_async_copy(scores_hbm.at[base + r0], row_a, sem_a).wait()
        pltpu.make_async_copy(scores_hbm.at[base + r0 + 1], row_b, sem_b).wait()
        process_pair(r0, row_a, row_b)

        @pl.when(q < rpw // 4 - 1)
        def _():
            pltpu.async_copy(scores_hbm.at[base + r0 + 4], row_a, sem_a)
            pltpu.async_copy(scores_hbm.at[base + r0 + 5], row_b, sem_b)

        pltpu.make_async_copy(scores_hbm.at[base + r0 + 2], row_c, sem_c).wait()
        pltpu.make_async_copy(scores_hbm.at[base + r0 + 3], row_d, sem_d).wait()
        process_pair(r0 + 2, row_c, row_d)
        return carry

    lax.fori_loop(0, rpw // 4, body4, 0)
    pltpu.sync_copy(idx_acc, idx_hbm.at[pl.ds(base, rpw)])
    pltpu.sync_copy(val_acc, val_hbm.at[pl.ds(base, rpw)])


def _topk(scores, cmax):
    nq = scores.shape[0]
    rpw = nq // NUM_WORKERS
    mesh = plsc.VectorSubcoreMesh(core_axis_name="c", subcore_axis_name="s")
    fn = pl.kernel(
        functools.partial(_topk_body, rpw),
        out_type=[jax.ShapeDtypeStruct((nq, TOPK), jnp.int32),
                  jax.ShapeDtypeStruct((nq, TOPK), jnp.float32)],
        mesh=mesh,
        compiler_params=pltpu.CompilerParams(needs_layout_passes=False),
        scratch_types=[pltpu.VMEM((NS,), jnp.float32),
                       pltpu.VMEM((NS,), jnp.float32),
                       pltpu.VMEM((NS,), jnp.float32),
                       pltpu.VMEM((NS,), jnp.float32),
                       pltpu.VMEM((rpw, NCHUNK), jnp.float32),
                       pltpu.VMEM((rpw, TOPK), jnp.int32),
                       pltpu.VMEM((rpw, TOPK), jnp.float32),
                       pltpu.SemaphoreType.DMA,
                       pltpu.SemaphoreType.DMA,
                       pltpu.SemaphoreType.DMA,
                       pltpu.SemaphoreType.DMA],
    )
    return fn(scores, cmax)


def kernel(query, slot_keys, reliability_mask, W_router):
    b, s, d = query.shape
    r = W_router.shape[0]
    scale = 1.0 / math.sqrt(r)
    q2 = query.reshape(b * s, d)
    wt = jnp.zeros((d, RPAD), jnp.float32).at[:, :r].set(W_router.T)
    rk = _project(slot_keys, wt)
    rkt = rk.T
    mask2d = reliability_mask.reshape(1, NS)
    gsz = (b * s) // NGROUPS
    outs = []
    for g in range(NGROUPS):
        sc_g, cm_g = _scores(q2[g * gsz:(g + 1) * gsz], wt, rkt, mask2d,
                             scale)
        outs.append(_topk(sc_g, cm_g))
    idx = jnp.concatenate([o[0] for o in outs])
    val = jnp.concatenate([o[1] for o in outs])
    return idx.reshape(b, s, TOPK), val.reshape(b, s, TOPK)
